# baseline (speedup 1.0000x reference)
import functools

import jax
import jax.numpy as jnp
from jax import lax
from jax.experimental import pallas as pl
from jax.experimental.pallas import tpu as pltpu
from jax.experimental.pallas import tpu_sc as plsc

_B = 16384
_D = 40
_NC = 2
_NS = 16
_NW = _NC * _NS
_BPW = _B // _NW
_CH = 128
_NCH = _BPW // _CH

_mesh = plsc.VectorSubcoreMesh(core_axis_name="c", subcore_axis_name="s")


@functools.partial(
    pl.kernel,
    out_type=[jax.ShapeDtypeStruct((_B, _D), jnp.float32)] * 4,
    mesh=_mesh,
    scratch_types=[
        pltpu.VMEM((_BPW,), jnp.int32),
        pltpu.VMEM((_BPW,), jnp.int32),
        pltpu.VMEM((_BPW, _D), jnp.float32),
        pltpu.VMEM((_BPW, _D), jnp.float32),
        pltpu.VMEM((_BPW, _D), jnp.float32),
        pltpu.VMEM((_BPW, _D), jnp.float32),
        pltpu.SemaphoreType.DMA,
        pltpu.SemaphoreType.DMA,
    ],
    compiler_params=pltpu.CompilerParams(use_tc_tiling_on_sc=False),
)
def _sc_gather(uid_hbm, iid_hbm, gu_hbm, gi_hbm, mu_hbm, mi_hbm,
               gu_out, gi_out, mu_out, mi_out,
               uid_v, iid_v, gu_v, gi_v, mu_v, mi_v, gsem, osem):
    wid = lax.axis_index("s") * _NC + lax.axis_index("c")
    base = wid * _BPW
    pltpu.sync_copy(uid_hbm.at[pl.ds(base, _BPW)], uid_v)
    pltpu.sync_copy(iid_hbm.at[pl.ds(base, _BPW)], iid_v)
    copies = []
    for c in range(_NCH):
        sl = pl.ds(c * _CH, _CH)
        for table, idx, buf in ((gu_hbm, uid_v, gu_v), (gi_hbm, iid_v, gi_v),
                                (mu_hbm, uid_v, mu_v), (mi_hbm, iid_v, mi_v)):
            copies.append(pltpu.async_copy(table.at[idx.at[sl]], buf.at[sl], gsem))
    for cp in copies:
        cp.wait()
    outs = []
    for buf, out in ((gu_v, gu_out), (gi_v, gi_out), (mu_v, mu_out), (mi_v, mi_out)):
        outs.append(pltpu.async_copy(buf, out.at[pl.ds(base, _BPW)], osem))
    for cp in outs:
        cp.wait()


_BLK = 2048


def _mlp_body(gu_ref, gi_ref, mu_ref, mi_ref, w1u_ref, w1i_ref, b1_ref,
              w2_ref, b2_ref, wng_ref, wnh_ref, bn_ref, out_ref):
    g = gu_ref[...] * gi_ref[...]
    h1 = jnp.dot(mu_ref[...], w1u_ref[...], preferred_element_type=jnp.float32)
    h1 = h1 + jnp.dot(mi_ref[...], w1i_ref[...], preferred_element_type=jnp.float32)
    h1 = jnp.maximum(h1 + b1_ref[...], 0.0)
    h2 = jnp.dot(h1, w2_ref[...], preferred_element_type=jnp.float32)
    h2 = jnp.maximum(h2 + b2_ref[...], 0.0)
    logit = (jnp.sum(g * wng_ref[...], axis=1, keepdims=True)
             + jnp.sum(h2 * wnh_ref[...], axis=1, keepdims=True)
             + bn_ref[...])
    out_ref[...] = 1.0 / (1.0 + jnp.exp(-logit))


def _mlp_call(gu, gi, mu, mi, w1u, w1i, b1, w2t, b2, wng, wnh, bn):
    grid = (_B // _BLK,)
    row_spec = pl.BlockSpec((_BLK, _D), lambda i: (i, 0))
    full = lambda shape: pl.BlockSpec(shape, lambda i: (0,) * len(shape))
    return pl.pallas_call(
        _mlp_body,
        grid=grid,
        in_specs=[
            row_spec, row_spec, row_spec, row_spec,
            full((_D, 20)), full((_D, 20)), full((1, 20)),
            full((20, 10)), full((1, 10)),
            full((1, _D)), full((1, 10)), full((1, 1)),
        ],
        out_specs=pl.BlockSpec((_BLK, 1), lambda i: (i, 0)),
        out_shape=jax.ShapeDtypeStruct((_B, 1), jnp.float32),
    )(gu, gi, mu, mi, w1u, w1i, b1, w2t, b2, wng, wnh, bn)


def kernel(batch, gmf_user, gmf_item, mlp_user, mlp_item, W1, b1, W2, b2, Wn, bn):
    uid = batch[:, 0]
    iid = batch[:, 1]
    gu, gi, mu, mi = _sc_gather(uid, iid, gmf_user, gmf_item, mlp_user, mlp_item)
    w1u = W1[:, :_D].T
    w1i = W1[:, _D:].T
    w2t = W2.T
    wng = Wn[:, :_D]
    wnh = Wn[:, _D:]
    out = _mlp_call(gu, gi, mu, mi, w1u, w1i, b1.reshape(1, 20), w2t,
                    b2.reshape(1, 10), wng, wnh, bn.reshape(1, 1))
    return out[:, 0]


# speedup vs baseline: 1.4032x; 1.4032x over previous
import functools

import jax
import jax.numpy as jnp
from jax import lax
from jax.experimental import pallas as pl
from jax.experimental.pallas import tpu as pltpu
from jax.experimental.pallas import tpu_sc as plsc

_B = 16384
_D = 40
_NC = 2
_NS = 16
_NW = _NC * _NS
_BPW = _B // _NW

_mesh = plsc.VectorSubcoreMesh(core_axis_name="c", subcore_axis_name="s")


@functools.partial(
    pl.kernel,
    out_type=[jax.ShapeDtypeStruct((_B, _D), jnp.float32)] * 4,
    mesh=_mesh,
    scratch_types=[
        pltpu.VMEM((_BPW,), jnp.int32),
        pltpu.VMEM((_BPW,), jnp.int32),
        pltpu.SemaphoreType.DMA,
        pltpu.SemaphoreType.DMA,
        pltpu.SemaphoreType.DMA,
        pltpu.SemaphoreType.DMA,
    ],
)
def _sc_gather(uid_hbm, iid_hbm, gu_hbm, gi_hbm, mu_hbm, mi_hbm,
               gu_out, gi_out, mu_out, mi_out,
               uid_s, iid_s, s0, s1, s2, s3):
    wid = lax.axis_index("s") * _NC + lax.axis_index("c")
    base = wid * _BPW
    pltpu.sync_copy(uid_hbm.at[pl.ds(base, _BPW)], uid_s)
    pltpu.sync_copy(iid_hbm.at[pl.ds(base, _BPW)], iid_s)

    def body(c, _):
        j0 = c * 16
        uv = uid_s[pl.ds(j0, 16)]
        iv = iid_s[pl.ds(j0, 16)]
        for l in range(16):
            ru = uv[l]
            ri = iv[l]
            j = base + j0 + l
            pltpu.async_copy(gu_hbm.at[pl.ds(ru, 1)], gu_out.at[pl.ds(j, 1)], s0)
            pltpu.async_copy(gi_hbm.at[pl.ds(ri, 1)], gi_out.at[pl.ds(j, 1)], s1)
            pltpu.async_copy(mu_hbm.at[pl.ds(ru, 1)], mu_out.at[pl.ds(j, 1)], s2)
            pltpu.async_copy(mi_hbm.at[pl.ds(ri, 1)], mi_out.at[pl.ds(j, 1)], s3)
        return 0

    lax.fori_loop(0, _BPW // 16, body, 0)
    for tab, out, sem in ((gu_hbm, gu_out, s0), (gi_hbm, gi_out, s1),
                          (mu_hbm, mu_out, s2), (mi_hbm, mi_out, s3)):
        pltpu.make_async_copy(tab.at[pl.ds(0, _BPW)],
                              out.at[pl.ds(base, _BPW)], sem).wait()


_BLK = 2048


def _mlp_body(gu_ref, gi_ref, mu_ref, mi_ref, w1u_ref, w1i_ref, b1_ref,
              w2_ref, b2_ref, wng_ref, wnh_ref, bn_ref, out_ref):
    g = gu_ref[...] * gi_ref[...]
    h1 = jnp.dot(mu_ref[...], w1u_ref[...], preferred_element_type=jnp.float32)
    h1 = h1 + jnp.dot(mi_ref[...], w1i_ref[...], preferred_element_type=jnp.float32)
    h1 = jnp.maximum(h1 + b1_ref[...], 0.0)
    h2 = jnp.dot(h1, w2_ref[...], preferred_element_type=jnp.float32)
    h2 = jnp.maximum(h2 + b2_ref[...], 0.0)
    logit = (jnp.sum(g * wng_ref[...], axis=1, keepdims=True)
             + jnp.sum(h2 * wnh_ref[...], axis=1, keepdims=True)
             + bn_ref[...])
    out_ref[...] = 1.0 / (1.0 + jnp.exp(-logit))


def _mlp_call(gu, gi, mu, mi, w1u, w1i, b1, w2t, b2, wng, wnh, bn):
    grid = (_B // _BLK,)
    row_spec = pl.BlockSpec((_BLK, _D), lambda i: (i, 0))
    full = lambda shape: pl.BlockSpec(shape, lambda i: (0,) * len(shape))
    return pl.pallas_call(
        _mlp_body,
        grid=grid,
        in_specs=[
            row_spec, row_spec, row_spec, row_spec,
            full((_D, 20)), full((_D, 20)), full((1, 20)),
            full((20, 10)), full((1, 10)),
            full((1, _D)), full((1, 10)), full((1, 1)),
        ],
        out_specs=pl.BlockSpec((_BLK, 1), lambda i: (i, 0)),
        out_shape=jax.ShapeDtypeStruct((_B, 1), jnp.float32),
    )(gu, gi, mu, mi, w1u, w1i, b1, w2t, b2, wng, wnh, bn)


def kernel(batch, gmf_user, gmf_item, mlp_user, mlp_item, W1, b1, W2, b2, Wn, bn):
    uid = batch[:, 0]
    iid = batch[:, 1]
    gu, gi, mu, mi = _sc_gather(uid, iid, gmf_user, gmf_item, mlp_user, mlp_item)
    w1u = W1[:, :_D].T
    w1i = W1[:, _D:].T
    w2t = W2.T
    wng = Wn[:, :_D]
    wnh = Wn[:, _D:]
    out = _mlp_call(gu, gi, mu, mi, w1u, w1i, b1.reshape(1, 20), w2t,
                    b2.reshape(1, 10), wng, wnh, bn.reshape(1, 1))
    return out[:, 0]
